# baseline (speedup 1.0000x reference)
import math

import jax
import jax.numpy as jnp
from jax.experimental import pallas as pl
from jax.experimental.pallas import tpu as pltpu

_NB = 32
_MD = 128
_BR = 128
_S = 2048

_WC = _S - 1
_WL = 4352


def _bias_bank():
    si = jax.lax.broadcasted_iota(jnp.int32, (8, _WL), 0)
    x = jax.lax.broadcasted_iota(jnp.int32, (8, _WL), 1)
    d = x - si - _WC
    rb = jnp.where(d > 0, _NB // 2, 0)
    a = jnp.abs(d)
    af = a.astype(jnp.float32)
    rp_if_large = _MD + jnp.log(af / _MD) / math.log(_MD / _NB) * (_NB - _MD)
    rp_if_large = jnp.minimum(rp_if_large, _MD - 1)
    large = rb.astype(jnp.float32) + rp_if_large
    small = (a + rb).astype(jnp.float32)
    out = jnp.where(a < _MD, small, large)
    return jnp.clip(out, 0, _NB - 1).astype(jnp.int32)


def _add_bias_kernel(x_ref, table_ref, o_ref, w_ref, bias_ref):
    r = pl.program_id(0)
    h = pl.program_id(1)

    @pl.when((h == 0) & (r == 0))
    def _():
        bucket = _bias_bank()
        acc = jnp.zeros((8, _WL), jnp.float32)
        for k in range(_NB):
            acc = jnp.where(bucket == k, table_ref[k, 0], acc)
        w_ref[...] = acc

    @pl.when(h == 0)
    def _():
        for g in range(_BR // 8):
            c = _WC - 8 * g
            rem = c % 128
            ba = (c - rem) - r * _BR
            chunk = w_ref[:, pl.ds(pl.multiple_of(ba, 128), _S + 128)]
            sliced = chunk[:, rem:rem + _S]
            bias_ref[8 * g:8 * g + 8, :] = sliced
            o_ref[:, 8 * g:8 * g + 8, :] = x_ref[:, 8 * g:8 * g + 8, :] + sliced[None]

    @pl.when(h != 0)
    def _():
        o_ref[...] = x_ref[...] + bias_ref[...]


def kernel(attention_scores, bias_table):
    b, h, s, _ = attention_scores.shape
    x = attention_scores.reshape(b * h, s, s)
    hb = 8
    grid = (s // _BR, (b * h) // hb)
    out = pl.pallas_call(
        _add_bias_kernel,
        grid=grid,
        in_specs=[
            pl.BlockSpec((hb, _BR, s), lambda r, hh: (hh, r, 0)),
            pl.BlockSpec((_NB, 1), lambda r, hh: (0, 0)),
        ],
        out_specs=pl.BlockSpec((hb, _BR, s), lambda r, hh: (hh, r, 0)),
        out_shape=jax.ShapeDtypeStruct((b * h, s, s), jnp.float32),
        scratch_shapes=[
            pltpu.VMEM((8, _WL), jnp.float32),
            pltpu.VMEM((_BR, s), jnp.float32),
        ],
        compiler_params=pltpu.CompilerParams(
            dimension_semantics=("parallel", "arbitrary")
        ),
    )(x, bias_table)
    return out.reshape(b, h, s, s)


# speedup vs baseline: 1.0008x; 1.0008x over previous
import math

import jax
import jax.numpy as jnp
from jax.experimental import pallas as pl
from jax.experimental.pallas import tpu as pltpu

_NB = 32
_MD = 128
_BR = 256
_S = 2048

_WC = _S - 1
_WL = 4352


def _bias_bank():
    si = jax.lax.broadcasted_iota(jnp.int32, (8, _WL), 0)
    x = jax.lax.broadcasted_iota(jnp.int32, (8, _WL), 1)
    d = x - si - _WC
    rb = jnp.where(d > 0, _NB // 2, 0)
    a = jnp.abs(d)
    af = a.astype(jnp.float32)
    rp_if_large = _MD + jnp.log(af / _MD) / math.log(_MD / _NB) * (_NB - _MD)
    rp_if_large = jnp.minimum(rp_if_large, _MD - 1)
    large = rb.astype(jnp.float32) + rp_if_large
    small = (a + rb).astype(jnp.float32)
    out = jnp.where(a < _MD, small, large)
    return jnp.clip(out, 0, _NB - 1).astype(jnp.int32)


def _add_bias_kernel(x_ref, table_ref, o_ref, w_ref, bias_ref):
    r = pl.program_id(0)
    h = pl.program_id(1)

    @pl.when((h == 0) & (r == 0))
    def _():
        bucket = _bias_bank()
        acc = jnp.zeros((8, _WL), jnp.float32)
        for k in range(_NB):
            acc = jnp.where(bucket == k, table_ref[k, 0], acc)
        w_ref[...] = acc

    @pl.when(h == 0)
    def _():
        for g in range(_BR // 8):
            c = _WC - 8 * g
            rem = c % 128
            ba = (c - rem) - r * _BR
            chunk = w_ref[:, pl.ds(pl.multiple_of(ba, 128), _S + 128)]
            sliced = chunk[:, rem:rem + _S]
            bias_ref[8 * g:8 * g + 8, :] = sliced
            o_ref[:, 8 * g:8 * g + 8, :] = x_ref[:, 8 * g:8 * g + 8, :] + sliced[None]

    @pl.when(h != 0)
    def _():
        o_ref[...] = x_ref[...] + bias_ref[...]


def kernel(attention_scores, bias_table):
    b, h, s, _ = attention_scores.shape
    x = attention_scores.reshape(b * h, s, s)
    hb = 4
    grid = (s // _BR, (b * h) // hb)
    out = pl.pallas_call(
        _add_bias_kernel,
        grid=grid,
        in_specs=[
            pl.BlockSpec((hb, _BR, s), lambda r, hh: (hh, r, 0)),
            pl.BlockSpec((_NB, 1), lambda r, hh: (0, 0)),
        ],
        out_specs=pl.BlockSpec((hb, _BR, s), lambda r, hh: (hh, r, 0)),
        out_shape=jax.ShapeDtypeStruct((b * h, s, s), jnp.float32),
        scratch_shapes=[
            pltpu.VMEM((8, _WL), jnp.float32),
            pltpu.VMEM((_BR, s), jnp.float32),
        ],
        compiler_params=pltpu.CompilerParams(
            dimension_semantics=("parallel", "arbitrary")
        ),
    )(x, bias_table)
    return out.reshape(b, h, s, s)
